# baseline (speedup 1.0000x reference)
import functools

import jax
import jax.numpy as jnp
from jax import lax
from jax.experimental import pallas as pl
from jax.experimental.pallas import tpu as pltpu
from jax.experimental.pallas import tpu_sc as plsc

N_NODES = 100000
NUM_SEGS = 256
EMB = 128
OUT = 10

NW = 32
BIG = 3128
SMALL = 3120
N_BIG = 20
TILE = 128
ETILE = 256
NE_TILES = 12
FULL_TILES = 24
TAIL_BIG = BIG - FULL_TILES * TILE
TAIL_SMALL = SMALL - FULL_TILES * TILE


def _seg_body(emb_hbm, idx_hbm, out_sum, out_cnt,
              ebuf, idx_flat, zb, cnt_s, idx_sm, cnt_v, acc, idx_sp,
              sem_in, sem_sc):
    cid = lax.axis_index("c")
    sid = lax.axis_index("s")
    w = cid * 16 + sid
    start = jnp.where(w < N_BIG, w * BIG,
                      N_BIG * BIG + (w - N_BIG) * SMALL).astype(jnp.int32)

    zeros16 = jnp.zeros((16,), jnp.float32)

    def fill_zb(i, _):
        zb[i // 8, pl.ds((i % 8) * 16, 16)] = zeros16
        return 0
    lax.fori_loop(0, 128, fill_zb, 0)

    def zero_cnt(i, _):
        cnt_s[i] = 0
        return 0
    lax.fori_loop(0, NUM_SEGS, zero_cnt, 0)


    pltpu.sync_copy(zb, acc.at[pl.ds(sid * 16, 16)])

    @pl.when(w < N_BIG)
    def _():
        pltpu.sync_copy(idx_hbm.at[pl.ds(start, BIG)],
                        idx_flat.at[pl.ds(0, BIG)])
        pltpu.sync_copy(idx_flat.at[pl.ds(0, BIG)],
                        idx_sp.at[pl.ds(start, BIG)])

    @pl.when(w >= N_BIG)
    def _():
        pltpu.sync_copy(idx_hbm.at[pl.ds(start, SMALL)],
                        idx_flat.at[pl.ds(0, SMALL)])
        pltpu.sync_copy(idx_flat.at[pl.ds(0, SMALL)],
                        idx_sp.at[pl.ds(start, SMALL)])

    plsc.subcore_barrier()

    def in_copy(j, slot):
        return pltpu.make_async_copy(
            emb_hbm.at[pl.ds(start + j * ETILE, ETILE)], ebuf.at[slot],
            sem_in)

    def sc_copies(j, slot):
        return [
            pltpu.make_async_copy(
                ebuf.at[slot, pl.ds(h * TILE, TILE)],
                acc.at[idx_flat.at[pl.ds(j * ETILE + h * TILE, TILE)]],
                sem_sc)
            for h in range(ETILE // TILE)
        ]

    in_copy(0, 0).start()

    def step(j, _):
        slot = lax.rem(j, 2)
        in_copy(j, slot).wait()

        @pl.when(j > 0)
        def _():
            for c in sc_copies(j - 1, 1 - slot):
                c.wait()

        @pl.when(j < NE_TILES - 1)
        def _():
            in_copy(j + 1, 1 - slot).start()

        for c in sc_copies(j, slot):
            c.start(add=True)
        return 0
    lax.fori_loop(0, NE_TILES, step, 0)

    for c in sc_copies(NE_TILES - 1, lax.rem(NE_TILES - 1, 2)):
        c.wait()

    tail = start + FULL_TILES * TILE
    n_tail = jnp.where(w < N_BIG, TAIL_BIG, TAIL_SMALL)

    @pl.when(w < N_BIG)
    def _():
        pltpu.sync_copy(emb_hbm.at[pl.ds(tail, TAIL_BIG)],
                        ebuf.at[0, pl.ds(0, TAIL_BIG)])
        pltpu.sync_copy(
            ebuf.at[0, pl.ds(0, TAIL_BIG)],
            acc.at[idx_flat.at[pl.ds(FULL_TILES * TILE, TAIL_BIG)]],
            add=True)

    @pl.when(w >= N_BIG)
    def _():
        pltpu.sync_copy(emb_hbm.at[pl.ds(tail, TAIL_SMALL)],
                        ebuf.at[0, pl.ds(0, TAIL_SMALL)])
        pltpu.sync_copy(
            ebuf.at[0, pl.ds(0, TAIL_SMALL)],
            acc.at[idx_flat.at[pl.ds(FULL_TILES * TILE, TAIL_SMALL)]],
            add=True)

    def ep(j, _):
        pltpu.sync_copy(idx_sp.at[pl.ds(start + j * TILE, TILE)], idx_sm)
        a = idx_sm[0]
        b = idx_sm[TILE - 1]

        @pl.when(a == b)
        def _():
            cnt_s[a] = cnt_s[a] + TILE

        @pl.when(a != b)
        def _():
            def hist(i, _):
                v = idx_sm[i]
                cnt_s[v] = cnt_s[v] + 1
                return 0
            lax.fori_loop(0, TILE, hist, 0)
        return 0
    lax.fori_loop(0, FULL_TILES, ep, 0)

    pltpu.sync_copy(idx_sp.at[pl.ds(tail, TILE)], idx_sm)
    ta = idx_sm[0]
    tb = idx_sm[n_tail - 1]

    @pl.when(ta == tb)
    def _():
        cnt_s[ta] = cnt_s[ta] + n_tail

    @pl.when(ta != tb)
    def _():
        def hist_tail(i, _):
            v = idx_sm[i]
            cnt_s[v] = cnt_s[v] + 1
            return 0
        lax.fori_loop(0, n_tail, hist_tail, 0)

    lane16 = lax.broadcasted_iota(jnp.int32, (16,), 0)

    def export_chunk(k, _):
        v = jnp.zeros((16,), jnp.int32)
        for l in range(16):
            v = jnp.where(lane16 == l, cnt_s[k * 16 + l], v)
        cnt_v[pl.ds(k * 16, 16)] = v
        return 0
    lax.fori_loop(0, 16, export_chunk, 0)

    pltpu.sync_copy(cnt_v, out_cnt.at[w])

    plsc.subcore_barrier()

    pltpu.sync_copy(acc.at[pl.ds(sid * 16, 16)],
                    out_sum.at[cid, pl.ds(sid * 16, 16)])


_seg_kernel = functools.partial(
    pl.kernel,
    out_type=[jax.ShapeDtypeStruct((2, NUM_SEGS, EMB), jnp.float32),
              jax.ShapeDtypeStruct((NW, NUM_SEGS), jnp.int32)],
    mesh=plsc.VectorSubcoreMesh(core_axis_name="c", subcore_axis_name="s",
                                num_cores=2, num_subcores=16),
    scratch_types=[
        pltpu.VMEM((2, ETILE, EMB), jnp.float32),
        pltpu.VMEM((BIG + 8,), jnp.int32),
        pltpu.VMEM((16, EMB), jnp.float32),
        pltpu.SMEM((NUM_SEGS,), jnp.int32),
        pltpu.SMEM((TILE,), jnp.int32),
        pltpu.VMEM((NUM_SEGS,), jnp.int32),
        pltpu.VMEM_SHARED((NUM_SEGS, EMB), jnp.float32),
        pltpu.VMEM_SHARED((N_NODES + 352,), jnp.int32),
        pltpu.SemaphoreType.DMA,
        pltpu.SemaphoreType.DMA,
    ],
)(_seg_body)


def _finish_body(ps_ref, pc_ref, w_ref, b_ref, o_ref):
    sums = ps_ref[0] + ps_ref[1]
    cnt = jnp.sum(pc_ref[...], axis=0).astype(jnp.float32)[:, None]
    mean = sums / jnp.maximum(cnt, 1.0)
    o_ref[...] = lax.dot_general(
        mean, w_ref[...], (((1,), (1,)), ((), ())),
        preferred_element_type=jnp.float32) + b_ref[...]


def kernel(node_emb, batch, W, b):
    idx = batch.astype(jnp.int32)
    ps, pc = _seg_kernel(node_emb, idx)
    return pl.pallas_call(
        _finish_body,
        out_shape=jax.ShapeDtypeStruct((NUM_SEGS, OUT), jnp.float32),
    )(ps, pc, W, b.reshape(1, OUT))


# speedup vs baseline: 1.2032x; 1.2032x over previous
import functools

import jax
import jax.numpy as jnp
from jax import lax
from jax.experimental import pallas as pl
from jax.experimental.pallas import tpu as pltpu
from jax.experimental.pallas import tpu_sc as plsc

N_NODES = 100000
NUM_SEGS = 256
EMB = 128
OUT = 10

NW = 32
BIG = 3128
SMALL = 3120
N_BIG = 20
TILE = 128
FULL_TILES = 24
TAIL_BIG = BIG - FULL_TILES * TILE
TAIL_SMALL = SMALL - FULL_TILES * TILE


def _seg_body(emb_hbm, idx_hbm, out_sum, out_cnt,
              ebuf, srow, idx_flat, zb, cnt_s, flg_s, idx_sm, cnt_v, acc,
              idx_sp, sem_in):
    cid = lax.axis_index("c")
    sid = lax.axis_index("s")
    w = cid * 16 + sid
    start = jnp.where(w < N_BIG, w * BIG,
                      N_BIG * BIG + (w - N_BIG) * SMALL).astype(jnp.int32)

    zeros16 = jnp.zeros((16,), jnp.float32)

    def fill_zb(i, _):
        zb[i // 8, pl.ds((i % 8) * 16, 16)] = zeros16
        return 0
    lax.fori_loop(0, 128, fill_zb, 0)

    def zero_cnt(i, _):
        cnt_s[i] = 0
        return 0
    lax.fori_loop(0, NUM_SEGS, zero_cnt, 0)


    pltpu.sync_copy(zb, acc.at[pl.ds(sid * 16, 16)])

    @pl.when(w < N_BIG)
    def _():
        pltpu.sync_copy(idx_hbm.at[pl.ds(start, BIG)],
                        idx_flat.at[pl.ds(0, BIG)])
        pltpu.sync_copy(idx_flat.at[pl.ds(0, BIG)],
                        idx_sp.at[pl.ds(start, BIG)])

    @pl.when(w >= N_BIG)
    def _():
        pltpu.sync_copy(idx_hbm.at[pl.ds(start, SMALL)],
                        idx_flat.at[pl.ds(0, SMALL)])
        pltpu.sync_copy(idx_flat.at[pl.ds(0, SMALL)],
                        idx_sp.at[pl.ds(start, SMALL)])

    tail = start + FULL_TILES * TILE
    n_tail = jnp.where(w < N_BIG, TAIL_BIG, TAIL_SMALL)

    def chk(j, _):
        pltpu.sync_copy(idx_sp.at[pl.ds(start + j * TILE, TILE)], idx_sm)
        a = idx_sm[0]
        b = idx_sm[TILE - 1]
        flg_s[j] = jnp.where(a == b, 1, 0)

        @pl.when(a == b)
        def _():
            cnt_s[a] = cnt_s[a] + TILE

        @pl.when(a != b)
        def _():
            def hist(i, _):
                v = idx_sm[i]
                cnt_s[v] = cnt_s[v] + 1
                return 0
            lax.fori_loop(0, TILE, hist, 0)
        return 0
    lax.fori_loop(0, FULL_TILES, chk, 0)

    pltpu.sync_copy(idx_sp.at[pl.ds(tail, TILE)], idx_sm)
    ta = idx_sm[0]
    tb = idx_sm[n_tail - 1]

    @pl.when(ta == tb)
    def _():
        cnt_s[ta] = cnt_s[ta] + n_tail

    @pl.when(ta != tb)
    def _():
        def hist_tail(i, _):
            v = idx_sm[i]
            cnt_s[v] = cnt_s[v] + 1
            return 0
        lax.fori_loop(0, n_tail, hist_tail, 0)

    plsc.subcore_barrier()

    def in_copy(j, slot):
        return pltpu.make_async_copy(
            emb_hbm.at[pl.ds(start + j * TILE, TILE)], ebuf.at[slot], sem_in)

    in_copy(0, 0).start()

    def step(j, _):
        slot = lax.rem(j, 2)
        in_copy(j, slot).wait()

        @pl.when(j < FULL_TILES - 1)
        def _():
            in_copy(j + 1, 1 - slot).start()

        cflag = flg_s[j] == 1

        @pl.when(cflag)
        def _():
            def vsum(r, carry):
                return tuple(carry[c] + ebuf[slot, r, pl.ds(c * 16, 16)]
                             for c in range(8))
            sums = lax.fori_loop(
                0, TILE, vsum,
                tuple(jnp.zeros((16,), jnp.float32) for _ in range(8)))
            for c in range(8):
                srow[0, pl.ds(c * 16, 16)] = sums[c]
            pltpu.sync_copy(srow, acc.at[idx_flat.at[pl.ds(j * TILE, 1)]],
                            add=True)

        @pl.when(jnp.logical_not(cflag))
        def _():
            pltpu.sync_copy(
                ebuf.at[slot], acc.at[idx_flat.at[pl.ds(j * TILE, TILE)]],
                add=True)
        return 0
    lax.fori_loop(0, FULL_TILES, step, 0)

    @pl.when(w < N_BIG)
    def _():
        pltpu.sync_copy(emb_hbm.at[pl.ds(tail, TAIL_BIG)],
                        ebuf.at[0, pl.ds(0, TAIL_BIG)])
        pltpu.sync_copy(
            ebuf.at[0, pl.ds(0, TAIL_BIG)],
            acc.at[idx_flat.at[pl.ds(FULL_TILES * TILE, TAIL_BIG)]],
            add=True)

    @pl.when(w >= N_BIG)
    def _():
        pltpu.sync_copy(emb_hbm.at[pl.ds(tail, TAIL_SMALL)],
                        ebuf.at[0, pl.ds(0, TAIL_SMALL)])
        pltpu.sync_copy(
            ebuf.at[0, pl.ds(0, TAIL_SMALL)],
            acc.at[idx_flat.at[pl.ds(FULL_TILES * TILE, TAIL_SMALL)]],
            add=True)

    lane16 = lax.broadcasted_iota(jnp.int32, (16,), 0)

    def export_chunk(k, _):
        v = jnp.zeros((16,), jnp.int32)
        for l in range(16):
            v = jnp.where(lane16 == l, cnt_s[k * 16 + l], v)
        cnt_v[pl.ds(k * 16, 16)] = v
        return 0
    lax.fori_loop(0, 16, export_chunk, 0)

    pltpu.sync_copy(cnt_v, out_cnt.at[w])

    plsc.subcore_barrier()

    pltpu.sync_copy(acc.at[pl.ds(sid * 16, 16)],
                    out_sum.at[cid, pl.ds(sid * 16, 16)])


_seg_kernel = functools.partial(
    pl.kernel,
    out_type=[jax.ShapeDtypeStruct((2, NUM_SEGS, EMB), jnp.float32),
              jax.ShapeDtypeStruct((NW, NUM_SEGS), jnp.int32)],
    mesh=plsc.VectorSubcoreMesh(core_axis_name="c", subcore_axis_name="s",
                                num_cores=2, num_subcores=16),
    scratch_types=[
        pltpu.VMEM((2, TILE, EMB), jnp.float32),
        pltpu.VMEM((1, EMB), jnp.float32),
        pltpu.VMEM((BIG + 8,), jnp.int32),
        pltpu.VMEM((16, EMB), jnp.float32),
        pltpu.SMEM((NUM_SEGS,), jnp.int32),
        pltpu.SMEM((FULL_TILES,), jnp.int32),
        pltpu.SMEM((TILE,), jnp.int32),
        pltpu.VMEM((NUM_SEGS,), jnp.int32),
        pltpu.VMEM_SHARED((NUM_SEGS, EMB), jnp.float32),
        pltpu.VMEM_SHARED((N_NODES + 352,), jnp.int32),
        pltpu.SemaphoreType.DMA,
    ],
)(_seg_body)


def _finish_body(ps_ref, pc_ref, w_ref, b_ref, o_ref):
    sums = ps_ref[0] + ps_ref[1]
    cnt = jnp.sum(pc_ref[...], axis=0).astype(jnp.float32)[:, None]
    mean = sums / jnp.maximum(cnt, 1.0)
    o_ref[...] = lax.dot_general(
        mean, w_ref[...], (((1,), (1,)), ((), ())),
        preferred_element_type=jnp.float32) + b_ref[...]


def kernel(node_emb, batch, W, b):
    idx = batch.astype(jnp.int32)
    ps, pc = _seg_kernel(node_emb, idx)
    return pl.pallas_call(
        _finish_body,
        out_shape=jax.ShapeDtypeStruct((NUM_SEGS, OUT), jnp.float32),
    )(ps, pc, W, b.reshape(1, OUT))
